# baseline (speedup 1.0000x reference)
import functools

import jax
import jax.numpy as jnp
from jax.experimental import pallas as pl
from jax.experimental.pallas import tpu as pltpu

_LANES = 128
_BU = 512
_NBUF = 4


def _build_block_diag(w_ref, a_ref):
    r = jax.lax.broadcasted_iota(jnp.int32, (_LANES, _LANES), 0)
    c = jax.lax.broadcasted_iota(jnp.int32, (_LANES, _LANES), 1)
    rm = r & 3
    cm = c & 3
    acc = jnp.zeros((_LANES, _LANES), jnp.float32)
    for o in range(4):
        for i in range(4):
            acc = acc + jnp.where((rm == o) & (cm == i), w_ref[o, i], 0.0)
    a_ref[...] = jnp.where((r >> 2) == (c >> 2), acc, 0.0)


def _apply_block_diag(a, xb):
    s = xb.shape[0]
    chunks = [
        jnp.dot(a, xb[c * _LANES:(c + 1) * _LANES, :],
                preferred_element_type=jnp.float32)
        for c in range(s // _LANES)
    ]
    return jnp.concatenate(chunks, axis=0)


def _make_stream_body(nt):
    def body(x_hbm, w_ref, o_hbm, xbuf, obuf, a_ref, insem, outsem):
        core = pl.program_id(0)
        base = core * nt

        _build_block_diag(w_ref, a_ref)

        def in_copy(t, slot):
            return pltpu.make_async_copy(
                x_hbm.at[pl.ds((base + t) * _BU, _BU)],
                xbuf.at[slot],
                insem.at[slot],
            )

        def out_copy(t, slot):
            return pltpu.make_async_copy(
                obuf.at[slot],
                o_hbm.at[pl.ds((base + t) * _BU, _BU)],
                outsem.at[slot],
            )

        for t0 in range(min(_NBUF, nt)):
            in_copy(t0, t0).start()

        def step(t, carry):
            slot = jax.lax.rem(t, _NBUF)
            in_copy(t, slot).wait()

            @pl.when(t >= _NBUF)
            def _drain():
                out_copy(t - _NBUF, slot).wait()

            xb = xbuf[slot].reshape(_BU * 8, _LANES)
            obuf[slot] = _apply_block_diag(a_ref[...], xb).reshape(
                _BU, 8, _LANES
            )
            out_copy(t, slot).start()

            @pl.when(t + _NBUF < nt)
            def _prefetch():
                in_copy(t + _NBUF, slot).start()

            return carry

        jax.lax.fori_loop(0, nt, step, 0)

        for t0 in range(min(_NBUF, nt)):
            t = nt - min(_NBUF, nt) + t0
            out_copy(t, jax.lax.rem(t, _NBUF)).wait()

    return body


def _narrow_kernel_body(x_ref, w_ref, o_ref):
    o_ref[...] = jax.lax.dot_general(
        x_ref[...],
        w_ref[...],
        dimension_numbers=(((1,), (1,)), ((), ())),
        preferred_element_type=jnp.float32,
    ).astype(o_ref.dtype)


def _narrow_path(x, weight):
    B, IN = x.shape
    bt = min(8192, B)
    Bg = pl.cdiv(B, 2 * bt) * 2 * bt
    x_p = x if Bg == B else jnp.pad(x, ((0, Bg - B), (0, 0)))
    nt = Bg // (2 * bt)
    y = pl.pallas_call(
        _narrow_kernel_body,
        out_shape=jax.ShapeDtypeStruct((Bg, IN), x.dtype),
        grid=(2, nt),
        in_specs=[
            pl.BlockSpec((bt, IN), lambda c, j, _nt=nt: (c * _nt + j, 0)),
            pl.BlockSpec(memory_space=pltpu.MemorySpace.VMEM),
        ],
        out_specs=pl.BlockSpec((bt, IN), lambda c, j, _nt=nt: (c * _nt + j, 0)),
        compiler_params=pltpu.CompilerParams(
            dimension_semantics=("parallel", "arbitrary"),
        ),
    )(x_p, weight)
    return y[:B]


@functools.partial(jax.jit, static_argnames=())
def kernel(x, weight):
    B, IN = x.shape
    rows_per_tile = 256 * _BU
    if IN != 4 or B % (2 * rows_per_tile) != 0:
        return _narrow_path(x, weight)

    nu = B // 256
    nt = nu // (2 * _BU)
    xv = x.reshape(B // 128, 128, 4).transpose(0, 2, 1).reshape(nu, 8, _LANES)

    yv = pl.pallas_call(
        _make_stream_body(nt),
        out_shape=jax.ShapeDtypeStruct((nu, 8, _LANES), x.dtype),
        grid=(2,),
        in_specs=[
            pl.BlockSpec(memory_space=pltpu.MemorySpace.HBM),
            pl.BlockSpec(memory_space=pltpu.MemorySpace.SMEM),
        ],
        out_specs=pl.BlockSpec(memory_space=pltpu.MemorySpace.HBM),
        scratch_shapes=[
            pltpu.VMEM((_NBUF, _BU, 8, _LANES), jnp.float32),
            pltpu.VMEM((_NBUF, _BU, 8, _LANES), jnp.float32),
            pltpu.VMEM((_LANES, _LANES), jnp.float32),
            pltpu.SemaphoreType.DMA((_NBUF,)),
            pltpu.SemaphoreType.DMA((_NBUF,)),
        ],
        compiler_params=pltpu.CompilerParams(
            dimension_semantics=("parallel",),
        ),
    )(xv, weight)

    return yv.reshape(B // 128, 4, 128).transpose(0, 2, 1).reshape(B, IN)


# speedup vs baseline: 1.0229x; 1.0229x over previous
import functools

import jax
import jax.numpy as jnp
from jax.experimental import pallas as pl
from jax.experimental.pallas import tpu as pltpu

_LANES = 128
_BU = 2048


def _build_block_diag(w_ref, a_ref):
    r = jax.lax.broadcasted_iota(jnp.int32, (_LANES, _LANES), 0)
    c = jax.lax.broadcasted_iota(jnp.int32, (_LANES, _LANES), 1)
    rm = r & 3
    cm = c & 3
    acc = jnp.zeros((_LANES, _LANES), jnp.float32)
    for o in range(4):
        for i in range(4):
            acc = acc + jnp.where((rm == o) & (cm == i), w_ref[o, i], 0.0)
    a_ref[...] = jnp.where((r >> 2) == (c >> 2), acc, 0.0)


def _packed_body(x_ref, w_ref, o_ref, a_ref):
    @pl.when(pl.program_id(1) == 0)
    def _init():
        _build_block_diag(w_ref, a_ref)

    s = _BU * 8
    xb = x_ref[...].reshape(s, _LANES)
    a = a_ref[...]
    chunks = [
        jnp.dot(a, xb[c * _LANES:(c + 1) * _LANES, :],
                preferred_element_type=jnp.float32)
        for c in range(s // _LANES)
    ]
    o_ref[...] = jnp.concatenate(chunks, axis=0).reshape(_BU, 8, _LANES)


def _narrow_kernel_body(x_ref, w_ref, o_ref):
    o_ref[...] = jax.lax.dot_general(
        x_ref[...],
        w_ref[...],
        dimension_numbers=(((1,), (1,)), ((), ())),
        preferred_element_type=jnp.float32,
    ).astype(o_ref.dtype)


def _narrow_path(x, weight):
    B, IN = x.shape
    bt = min(8192, B)
    Bg = pl.cdiv(B, 2 * bt) * 2 * bt
    x_p = x if Bg == B else jnp.pad(x, ((0, Bg - B), (0, 0)))
    nt = Bg // (2 * bt)
    y = pl.pallas_call(
        _narrow_kernel_body,
        out_shape=jax.ShapeDtypeStruct((Bg, IN), x.dtype),
        grid=(2, nt),
        in_specs=[
            pl.BlockSpec((bt, IN), lambda c, j, _nt=nt: (c * _nt + j, 0)),
            pl.BlockSpec(memory_space=pltpu.MemorySpace.VMEM),
        ],
        out_specs=pl.BlockSpec((bt, IN), lambda c, j, _nt=nt: (c * _nt + j, 0)),
        compiler_params=pltpu.CompilerParams(
            dimension_semantics=("parallel", "arbitrary"),
        ),
    )(x_p, weight)
    return y[:B]


@functools.partial(jax.jit, static_argnames=())
def kernel(x, weight):
    B, IN = x.shape
    rows_per_tile = 256 * _BU
    if IN != 4 or B % (2 * rows_per_tile) != 0:
        return _narrow_path(x, weight)

    nu = B // 256
    nt = nu // (2 * _BU)
    xv = x.reshape(B // 128, 128, 4).transpose(0, 2, 1).reshape(nu, 8, _LANES)

    yv = pl.pallas_call(
        _packed_body,
        out_shape=jax.ShapeDtypeStruct((nu, 8, _LANES), x.dtype),
        grid=(2, nt),
        in_specs=[
            pl.BlockSpec((_BU, 8, _LANES),
                         lambda c, j, _nt=nt: (c * _nt + j, 0, 0)),
            pl.BlockSpec(memory_space=pltpu.MemorySpace.SMEM),
        ],
        out_specs=pl.BlockSpec((_BU, 8, _LANES),
                               lambda c, j, _nt=nt: (c * _nt + j, 0, 0)),
        scratch_shapes=[pltpu.VMEM((_LANES, _LANES), jnp.float32)],
        compiler_params=pltpu.CompilerParams(
            dimension_semantics=("parallel", "arbitrary"),
        ),
    )(xv, weight)

    return yv.reshape(B // 128, 4, 128).transpose(0, 2, 1).reshape(B, IN)
